# Initial kernel scaffold; baseline (speedup 1.0000x reference)
import jax
import jax.numpy as jnp
from jax.experimental import pallas as pl


def kernel(nodes_atoms, nodes_bonds, nodes_monosacchs, batch_ids_atoms, batch_ids_bonds, batch_ids_monosacchs):
    raise NotImplementedError("write your pallas kernel here")


# speedup vs baseline: 6.9515x; 6.9515x over previous
import functools

import jax
import jax.numpy as jnp
from jax import lax
from jax.experimental import pallas as pl
from jax.experimental.pallas import tpu as pltpu
from jax.experimental.pallas import tpu_sc as plsc

NUM_GRAPHS = 512
D = 128
N_ATOMS = 400000
N_BONDS = 400000
N_MONO = 40000

NC = 2
NS = 16
NW = NC * NS

BLK = 320
IDW = 40
IDR = BLK // IDW

_mesh = plsc.VectorSubcoreMesh(
    core_axis_name="c", subcore_axis_name="s", num_cores=NC, num_subcores=NS
)


def _accumulate_body(x_a, x_b, x_m, ids_a, ids_b, ids_m, psum, pcnt,
                     xbuf, idbuf, ones, zbuf, zcnt, accum, csh):
    cid = lax.axis_index("c")
    sid = lax.axis_index("s")
    tid = sid * NC + cid

    zero16 = jnp.zeros((16,), jnp.float32)
    one16 = jnp.ones((16,), jnp.float32)

    for i in range(32):
        for j in range(D // 16):
            zbuf[i, pl.ds(j * 16, 16)] = zero16
        zcnt[i] = zero16
    pltpu.sync_copy(zbuf, accum.at[pl.ds(sid * 32, 32)])
    pltpu.sync_copy(zcnt, csh.at[pl.ds(sid * 32, 32)])
    for i in range(IDW):
        ones[i] = one16
    plsc.subcore_barrier()

    def do_array(x_hbm, ids_hbm, nblocks):
        base = nblocks // NW
        extra = nblocks % NW
        nj = base + jnp.where(tid < extra, 1, 0)
        start = tid * base + jnp.minimum(tid, extra)

        def body(j, carry):
            b = start + j
            pltpu.sync_copy(x_hbm.at[pl.ds(b * BLK, BLK)], xbuf)
            pltpu.sync_copy(ids_hbm.at[pl.ds(b * IDR, IDR)], idbuf)
            for i in range(IDR):
                pltpu.sync_copy(
                    xbuf.at[pl.ds(i * IDW, IDW)],
                    accum.at[idbuf.at[i]], add=True)
                pltpu.sync_copy(ones, csh.at[idbuf.at[i]], add=True)
            return carry

        lax.fori_loop(0, nj, body, 0)

    do_array(x_a, ids_a, N_ATOMS // BLK)
    do_array(x_b, ids_b, N_BONDS // BLK)
    do_array(x_m, ids_m, N_MONO // BLK)

    plsc.subcore_barrier()

    pltpu.sync_copy(accum.at[pl.ds(sid * 32, 32)], zbuf)
    pltpu.sync_copy(zbuf, psum.at[cid, pl.ds(sid * 32, 32)])
    pltpu.sync_copy(csh.at[pl.ds(sid * 32, 32)], zcnt)
    pltpu.sync_copy(zcnt, pcnt.at[cid, pl.ds(sid * 32, 32)])


_accumulate = pl.kernel(
    _accumulate_body,
    out_type=(
        jax.ShapeDtypeStruct((NC, NUM_GRAPHS, D), jnp.float32),
        jax.ShapeDtypeStruct((NC, NUM_GRAPHS, 16), jnp.float32),
    ),
    mesh=_mesh,
    compiler_params=pltpu.CompilerParams(
        use_tc_tiling_on_sc=False, needs_layout_passes=False),
    scratch_types=[
        pltpu.VMEM((BLK, D), jnp.float32),
        pltpu.VMEM((IDR, IDW), jnp.int32),
        pltpu.VMEM((IDW, 16), jnp.float32),
        pltpu.VMEM((32, D), jnp.float32),
        pltpu.VMEM((32, 16), jnp.float32),
        pltpu.VMEM_SHARED((NUM_GRAPHS, D), jnp.float32),
        pltpu.VMEM_SHARED((NUM_GRAPHS, 16), jnp.float32),
    ],
)


def _finalize_body(psum, pcnt, out, b0, b1, c0, c1, obuf):
    cid = lax.axis_index("c")
    sid = lax.axis_index("s")
    tid = sid * NC + cid
    rows = NUM_GRAPHS // NW
    base = tid * rows

    pltpu.sync_copy(psum.at[0, pl.ds(base, rows)], b0)
    pltpu.sync_copy(psum.at[1, pl.ds(base, rows)], b1)
    pltpu.sync_copy(pcnt.at[0, pl.ds(base, rows)], c0)
    pltpu.sync_copy(pcnt.at[1, pl.ds(base, rows)], c1)

    for s in range(rows):
        cnt = c0[s] + c1[s]
        recip = 1.0 / jnp.maximum(cnt, 1.0)
        for ch in range(D // 16):
            v = b0[s, pl.ds(ch * 16, 16)] + b1[s, pl.ds(ch * 16, 16)]
            obuf[s, pl.ds(ch * 16, 16)] = v * recip

    pltpu.sync_copy(obuf, out.at[pl.ds(base, rows)])


_finalize = pl.kernel(
    _finalize_body,
    out_type=jax.ShapeDtypeStruct((NUM_GRAPHS, D), jnp.float32),
    mesh=_mesh,
    compiler_params=pltpu.CompilerParams(
        use_tc_tiling_on_sc=False, needs_layout_passes=False),
    scratch_types=[
        pltpu.VMEM((NUM_GRAPHS // NW, D), jnp.float32),
        pltpu.VMEM((NUM_GRAPHS // NW, D), jnp.float32),
        pltpu.VMEM((NUM_GRAPHS // NW, 16), jnp.float32),
        pltpu.VMEM((NUM_GRAPHS // NW, 16), jnp.float32),
        pltpu.VMEM((NUM_GRAPHS // NW, D), jnp.float32),
    ],
)


@jax.jit
def kernel(nodes_atoms, nodes_bonds, nodes_monosacchs,
           batch_ids_atoms, batch_ids_bonds, batch_ids_monosacchs):
    ids_a = batch_ids_atoms.astype(jnp.int32).reshape(N_ATOMS // IDW, IDW)
    ids_b = batch_ids_bonds.astype(jnp.int32).reshape(N_BONDS // IDW, IDW)
    ids_m = batch_ids_monosacchs.astype(jnp.int32).reshape(N_MONO // IDW, IDW)
    psum, pcnt = _accumulate(nodes_atoms, nodes_bonds, nodes_monosacchs,
                             ids_a, ids_b, ids_m)
    return _finalize(psum, pcnt)


# speedup vs baseline: 12.9748x; 1.8665x over previous
import jax
import jax.numpy as jnp
from jax import lax
from jax.experimental import pallas as pl
from jax.experimental.pallas import tpu as pltpu
from jax.experimental.pallas import tpu_sc as plsc

NUM_GRAPHS = 512
D = 128
N_ATOMS = 400000
N_BONDS = 400000
N_MONO = 40000

NC = 2
NS = 16
NW = NC * NS

BLK = 400
IDW = 80
IDR = BLK // IDW

_params = pltpu.CompilerParams(
    use_tc_tiling_on_sc=False, needs_layout_passes=False)

_mesh = plsc.VectorSubcoreMesh(
    core_axis_name="c", subcore_axis_name="s", num_cores=NC, num_subcores=NS
)


def _accumulate_body(x_a, x_b, x_m, ids_a, ids_b, ids_m, psum, pcnt,
                     xbuf, idbuf, cnt, ibuf, zbuf, zcnt, gsem, accum, csh):
    cid = lax.axis_index("c")
    sid = lax.axis_index("s")
    tid = sid * NC + cid

    zero16 = jnp.zeros((16,), jnp.float32)
    one16 = jnp.ones((16,), jnp.float32)
    iota16 = lax.iota(jnp.int32, 16)

    for i in range(32):
        for j in range(D // 16):
            zbuf[i, pl.ds(j * 16, 16)] = zero16
    for i in range(2):
        zcnt[i] = zero16
    for i in range(32):
        cnt[i] = zero16
    ibuf[0] = iota16
    ibuf[1] = iota16 + 16
    pltpu.sync_copy(zbuf, accum.at[pl.ds(sid * 32, 32)])
    pltpu.sync_copy(zcnt, csh.at[pl.ds(sid * 2, 2)])
    plsc.subcore_barrier()

    def issue(x_hbm, ids_hbm, b, slot):
        pltpu.async_copy(x_hbm.at[pl.ds(b * BLK, BLK)], xbuf.at[slot],
                         gsem.at[slot])
        pltpu.async_copy(ids_hbm.at[pl.ds(b * IDR, IDR)], idbuf.at[slot],
                         gsem.at[slot])

    def drain(x_hbm, ids_hbm, slot):
        pltpu.make_async_copy(x_hbm.at[pl.ds(0, BLK)], xbuf.at[slot],
                              gsem.at[slot]).wait()
        pltpu.make_async_copy(ids_hbm.at[pl.ds(0, IDR)], idbuf.at[slot],
                              gsem.at[slot]).wait()

    def do_array(x_hbm, ids_hbm, nblocks):
        base = nblocks // NW
        extra = nblocks % NW
        nj = base + jnp.where(tid < extra, 1, 0)
        start = tid * base + jnp.minimum(tid, extra)

        issue(x_hbm, ids_hbm, start, 0)

        def body(j, carry):
            slot = j % 2

            @pl.when(j + 1 < nj)
            def _():
                issue(x_hbm, ids_hbm, start + j + 1, 1 - slot)

            drain(x_hbm, ids_hbm, slot)

            for r in range(IDR):
                for k in range(IDW // 16):
                    ids_v = idbuf[slot, r, pl.ds(k * 16, 16)]
                    plsc.addupdate_scatter(
                        cnt, [lax.shift_right_logical(ids_v, 4),
                              lax.bitwise_and(ids_v, 15)], one16)

            for i in range(IDR):
                pltpu.sync_copy(
                    xbuf.at[slot, pl.ds(i * IDW, IDW)],
                    accum.at[idbuf.at[slot, i]], add=True)
            return carry

        lax.fori_loop(0, nj, body, 0)

    do_array(x_a, ids_a, N_ATOMS // BLK)
    do_array(x_b, ids_b, N_BONDS // BLK)
    do_array(x_m, ids_m, N_MONO // BLK)

    pltpu.sync_copy(cnt.at[pl.ds(0, 16)], csh.at[ibuf.at[0]], add=True)
    pltpu.sync_copy(cnt.at[pl.ds(16, 16)], csh.at[ibuf.at[1]], add=True)

    plsc.subcore_barrier()

    pltpu.sync_copy(accum.at[pl.ds(sid * 32, 32)], zbuf)
    pltpu.sync_copy(zbuf, psum.at[cid, pl.ds(sid * 32, 32)])
    pltpu.sync_copy(csh.at[pl.ds(sid * 2, 2)], zcnt)
    pltpu.sync_copy(zcnt, pcnt.at[cid, pl.ds(sid * 2, 2)])


_accumulate = pl.kernel(
    _accumulate_body,
    out_type=(
        jax.ShapeDtypeStruct((NC, NUM_GRAPHS, D), jnp.float32),
        jax.ShapeDtypeStruct((NC, 32, 16), jnp.float32),
    ),
    mesh=_mesh,
    compiler_params=_params,
    scratch_types=[
        pltpu.VMEM((2, BLK, D), jnp.float32),
        pltpu.VMEM((2, IDR, IDW), jnp.int32),
        pltpu.VMEM((32, 16), jnp.float32),
        pltpu.VMEM((2, 16), jnp.int32),
        pltpu.VMEM((32, D), jnp.float32),
        pltpu.VMEM((2, 16), jnp.float32),
        pltpu.SemaphoreType.DMA((2,)),
        pltpu.VMEM_SHARED((NUM_GRAPHS, D), jnp.float32),
        pltpu.VMEM_SHARED((32, 16), jnp.float32),
    ],
)


def _finalize_body(psum, pcnt, out, b0, b1, c0, c1, obuf):
    cid = lax.axis_index("c")
    sid = lax.axis_index("s")
    tid = sid * NC + cid
    rows = NUM_GRAPHS // NW
    base = tid * rows

    pltpu.sync_copy(psum.at[0, pl.ds(base, rows)], b0)
    pltpu.sync_copy(psum.at[1, pl.ds(base, rows)], b1)
    pltpu.sync_copy(pcnt.at[0, tid], c0)
    pltpu.sync_copy(pcnt.at[1, tid], c1)

    cv = c0[...] + c1[...]
    rv = 1.0 / jnp.maximum(cv, 1.0)
    iota16 = lax.iota(jnp.int32, 16)
    for s in range(rows):
        rs = jnp.sum(jnp.where(iota16 == s, rv, 0.0))
        recip = lax.broadcast_in_dim(rs, (16,), ())
        for ch in range(D // 16):
            v = b0[s, pl.ds(ch * 16, 16)] + b1[s, pl.ds(ch * 16, 16)]
            obuf[s, pl.ds(ch * 16, 16)] = v * recip

    pltpu.sync_copy(obuf, out.at[pl.ds(base, rows)])


_finalize = pl.kernel(
    _finalize_body,
    out_type=jax.ShapeDtypeStruct((NUM_GRAPHS, D), jnp.float32),
    mesh=_mesh,
    compiler_params=_params,
    scratch_types=[
        pltpu.VMEM((NUM_GRAPHS // NW, D), jnp.float32),
        pltpu.VMEM((NUM_GRAPHS // NW, D), jnp.float32),
        pltpu.VMEM((16,), jnp.float32),
        pltpu.VMEM((16,), jnp.float32),
        pltpu.VMEM((NUM_GRAPHS // NW, D), jnp.float32),
    ],
)


@jax.jit
def kernel(nodes_atoms, nodes_bonds, nodes_monosacchs,
           batch_ids_atoms, batch_ids_bonds, batch_ids_monosacchs):
    ids_a = batch_ids_atoms.astype(jnp.int32).reshape(N_ATOMS // IDW, IDW)
    ids_b = batch_ids_bonds.astype(jnp.int32).reshape(N_BONDS // IDW, IDW)
    ids_m = batch_ids_monosacchs.astype(jnp.int32).reshape(N_MONO // IDW, IDW)
    psum, pcnt = _accumulate(nodes_atoms, nodes_bonds, nodes_monosacchs,
                             ids_a, ids_b, ids_m)
    return _finalize(psum, pcnt)
